# baseline (speedup 1.0000x reference)
import functools

import jax
import jax.numpy as jnp
from jax import lax
from jax.experimental import pallas as pl
from jax.experimental.pallas import tpu as pltpu
from jax.experimental.pallas import tpu_sc as plsc

B = 2
L = 2048
C = 1024
K = 3
N = B * L
NW = 32
RPW = N // NW
RCHUNK = 8
NCHUNK = RPW // RCHUNK
G = 2 * K


def _prep_kernel(x_ref, w_ref, bias_ref, meta_ref, xpk_ref):
    xf = x_ref[...]
    s = jnp.dot(xf, w_ref[...], preferred_element_type=jnp.float32)

    z8 = jnp.zeros((1, 8), jnp.float32)
    sm1 = jnp.concatenate([z8, s[:-1, 0:8]], axis=0)
    sp1 = jnp.concatenate([s[1:, 16:24], z8], axis=0)

    row = lax.broadcasted_iota(jnp.int32, (N, 1), 0)
    l2d = jnp.bitwise_and(row, L - 1)
    sm1 = jnp.where(l2d != 0, sm1, 0.0)
    sp1 = jnp.where(l2d != L - 1, sp1, 0.0)
    y8 = sm1 + s[:, 8:16] + sp1 + bias_ref[...]

    off = y8[:, 0:3]
    m = jax.nn.sigmoid(y8[:, 3:6])
    lf = l2d.astype(jnp.float32)
    bb = row - l2d
    pos = jnp.clip(lf + off, 0.0, float(L - 1))
    fp = jnp.floor(pos)
    alpha = pos - fp
    fpi = fp.astype(jnp.int32)
    cpi = jnp.minimum(fpi + 1, L - 1)
    idx6 = jnp.concatenate([fpi + bb, cpi + bb], axis=1)
    w6 = jnp.concatenate([m * (1.0 - alpha), m * alpha], axis=1)
    wq = (w6 * 1048575.0 + 0.5).astype(jnp.int32)
    meta6 = jnp.bitwise_or(idx6, lax.shift_left(wq, 12))
    zi2 = jnp.zeros((N, 2), jnp.int32)
    meta_ref[...] = jnp.concatenate([meta6, zi2], axis=1)
    bits16 = lax.bitcast_convert_type(xf.astype(jnp.bfloat16), jnp.int16)
    b32 = bits16.astype(jnp.int32)
    lo = jnp.bitwise_and(b32[:, : C // 2], 0xFFFF)
    hi = lax.shift_left(b32[:, C // 2:], 16)
    xpk_ref[...] = jnp.bitwise_or(hi, lo)


def _sc_gather_kernel(x_hbm, meta_hbm, out_hbm,
                      meta_v, idxc_v, rows_v, out_v, gsem, osem):
    wid = lax.axis_index("s") * 2 + lax.axis_index("c")
    base = wid * RPW

    pltpu.sync_copy(meta_hbm.at[pl.ds(base, RPW), :], meta_v)

    lane16 = lax.iota(jnp.int32, 16)
    rowpat = []
    colpat = []
    for k in range(RCHUNK * G // 16):
        e = lane16 + 16 * k
        rr = lax.div(e, jnp.int32(G))
        rowpat.append(rr)
        colpat.append(e - rr * G)

    def meta_chunk(c):
        return [plsc.load_gather(meta_v, [rowpat[k] + c * RCHUNK, colpat[k]])
                for k in range(RCHUNK * G // 16)]

    def build_idx(c, buf):
        for k, mv in enumerate(meta_chunk(c)):
            idxc_v[buf, pl.ds(16 * k, 16)] = jnp.bitwise_and(mv, 0xFFF)

    def gather_desc(c, buf):
        return pltpu.make_async_copy(
            x_hbm.at[idxc_v.at[buf]], rows_v.at[buf], gsem.at[buf])

    def out_desc(c, buf):
        return pltpu.make_async_copy(
            out_v.at[buf], out_hbm.at[pl.ds(base + c * RCHUNK, RCHUNK)],
            osem.at[buf])

    build_idx(0, 0)
    gather_desc(0, 0).start()

    def do_chunk(c, buf):
        gather_desc(c, buf).wait()

        @pl.when(c + 1 < NCHUNK)
        def _():
            build_idx(c + 1, 1 - buf)
            gather_desc(c + 1, 1 - buf).start()

        @pl.when(c >= 2)
        def _():
            out_desc(c - 2, buf).wait()

        wscale = jnp.float32(1.0 / 1048575.0)
        wgrp = [lax.convert_element_type(
                    lax.shift_right_logical(mv, 12), jnp.float32) * wscale
                for mv in meta_chunk(c)]
        for r in range(RCHUNK):
            def _w(j, r=r):
                pos = r * G + j
                return wgrp[pos // 16][pos % 16]
            ws = [_w(j) for j in range(G)]

            def ch(i, _, buf=buf, r=r, ws=ws):
                sl = pl.ds(i * 16, 16)
                acc_lo = None
                acc_hi = None
                for j in range(G):
                    v = rows_v[buf, r * G + j, sl]
                    f_lo = plsc.bitcast(lax.shift_left(v, 16), jnp.float32)
                    f_hi = plsc.bitcast(
                        jnp.bitwise_and(v, jnp.int32(-65536)), jnp.float32)
                    if acc_lo is None:
                        acc_lo = f_lo * ws[j]
                        acc_hi = f_hi * ws[j]
                    else:
                        acc_lo += f_lo * ws[j]
                        acc_hi += f_hi * ws[j]
                out_v[buf, r, sl] = acc_lo
                out_v[buf, r, pl.ds(C // 2 + i * 16, 16)] = acc_hi
                return 0

            lax.fori_loop(0, C // 32, ch, 0)
        out_desc(c, buf).start()

    def pair(pp, carry):
        do_chunk(2 * pp, 0)
        do_chunk(2 * pp + 1, 1)
        return carry

    lax.fori_loop(0, NCHUNK // 2, pair, 0)
    out_desc(NCHUNK - 2, 0).wait()
    out_desc(NCHUNK - 1, 1).wait()


def _prep(x2d, w_all, bias):
    return pl.pallas_call(
        _prep_kernel,
        out_shape=(
            jax.ShapeDtypeStruct((N, G + 2), jnp.int32),
            jax.ShapeDtypeStruct((N, C // 2), jnp.int32),
        ),
    )(x2d, w_all, bias)


@functools.cache
def _make_sc_gather():
    return pl.kernel(
        _sc_gather_kernel,
        out_type=jax.ShapeDtypeStruct((N, C), jnp.float32),
        mesh=plsc.VectorSubcoreMesh(core_axis_name="c", subcore_axis_name="s"),
        scratch_types=[
            pltpu.VMEM((RPW, G + 2), jnp.int32),
            pltpu.VMEM((2, RCHUNK * G), jnp.int32),
            pltpu.VMEM((2, RCHUNK * G, C // 2), jnp.int32),
            pltpu.VMEM((2, RCHUNK, C), jnp.float32),
            pltpu.SemaphoreType.DMA((2,)),
            pltpu.SemaphoreType.DMA((2,)),
        ],
        compiler_params=pltpu.CompilerParams(needs_layout_passes=False),
    )


def kernel(x, W_off, b_off, W_mask, b_mask):
    x2d = x.reshape(N, C)
    z2 = jnp.zeros((C, 2), jnp.float32)
    blocks = [
        jnp.concatenate(
            [W_off[:, :, d].T, W_mask[:, :, d].T, z2], axis=1)
        for d in range(K)
    ]
    w_all = jnp.concatenate(blocks, axis=1)
    zb = jnp.zeros((2,), jnp.float32)
    bias = jnp.concatenate([b_off, b_mask, zb]).reshape(1, 2 * K + 2)

    meta, xpk = _prep(x2d, w_all, bias)
    out2d = _make_sc_gather()(xpk, meta)
    return out2d.reshape(B, L, C).transpose(0, 2, 1).reshape(B, L, C)
